# baseline (speedup 1.0000x reference)
import jax
import jax.numpy as jnp
from jax.experimental import pallas as pl
from jax.experimental.pallas import tpu as pltpu

BN = 8192


def _proj_kernel(x_ref, w_ref, b_ref, out_ref):
    x = x_ref[...]
    w = w_ref[...]
    acc = jax.lax.dot_general(
        x, w, (((1,), (1,)), ((), ())), preferred_element_type=jnp.float32
    )
    out_ref[...] = acc + b_ref[...]


def kernel(inputs, labels, W, b):
    batch, nhid = inputs.shape
    ntokens = W.shape[0]
    b2 = b.reshape(1, ntokens)
    logits = pl.pallas_call(
        _proj_kernel,
        grid=(pl.cdiv(ntokens, BN),),
        in_specs=[
            pl.BlockSpec((batch, nhid), lambda i: (0, 0)),
            pl.BlockSpec((BN, nhid), lambda i: (i, 0)),
            pl.BlockSpec((1, BN), lambda i: (0, i)),
        ],
        out_specs=pl.BlockSpec((batch, BN), lambda i: (0, i)),
        out_shape=jax.ShapeDtypeStruct((batch, ntokens), jnp.float32),
        compiler_params=pltpu.CompilerParams(
            dimension_semantics=("arbitrary",),
        ),
    )(inputs, W, b2)
    return (logits, labels)


# speedup vs baseline: 1.1204x; 1.1204x over previous
import jax
import jax.numpy as jnp
from jax.experimental import pallas as pl
from jax.experimental.pallas import tpu as pltpu

BN = 16384


def _proj_kernel(x_ref, w_ref, b_ref, out_ref):
    x = x_ref[...]
    w = w_ref[...]
    acc = jax.lax.dot_general(
        x, w, (((1,), (1,)), ((), ())), preferred_element_type=jnp.float32
    )
    out_ref[...] = acc + b_ref[...]


def kernel(inputs, labels, W, b):
    batch, nhid = inputs.shape
    ntokens = W.shape[0]
    b2 = b.reshape(1, ntokens)
    logits = pl.pallas_call(
        _proj_kernel,
        grid=(pl.cdiv(ntokens, BN),),
        in_specs=[
            pl.BlockSpec((batch, nhid), lambda i: (0, 0)),
            pl.BlockSpec((BN, nhid), lambda i: (i, 0)),
            pl.BlockSpec((1, BN), lambda i: (0, i)),
        ],
        out_specs=pl.BlockSpec((batch, BN), lambda i: (0, i)),
        out_shape=jax.ShapeDtypeStruct((batch, ntokens), jnp.float32),
        compiler_params=pltpu.CompilerParams(
            dimension_semantics=("arbitrary",),
        ),
    )(inputs, W, b2)
    return (logits, labels)


# speedup vs baseline: 1.1374x; 1.0152x over previous
import jax
import jax.numpy as jnp
from jax.experimental import pallas as pl
from jax.experimental.pallas import tpu as pltpu

BN = 32768


def _proj_kernel(x_ref, w_ref, b_ref, out_ref):
    x = x_ref[...]
    w = w_ref[...]
    acc = jax.lax.dot_general(
        x, w, (((1,), (1,)), ((), ())), preferred_element_type=jnp.float32
    )
    out_ref[...] = acc + b_ref[...]


def kernel(inputs, labels, W, b):
    batch, nhid = inputs.shape
    ntokens = W.shape[0]
    b2 = b.reshape(1, ntokens)
    logits = pl.pallas_call(
        _proj_kernel,
        grid=(pl.cdiv(ntokens, BN),),
        in_specs=[
            pl.BlockSpec((batch, nhid), lambda i: (0, 0)),
            pl.BlockSpec((BN, nhid), lambda i: (i, 0)),
            pl.BlockSpec((1, BN), lambda i: (0, i)),
        ],
        out_specs=pl.BlockSpec((batch, BN), lambda i: (0, i)),
        out_shape=jax.ShapeDtypeStruct((batch, ntokens), jnp.float32),
        compiler_params=pltpu.CompilerParams(
            dimension_semantics=("arbitrary",),
        ),
    )(inputs, W, b2)
    return (logits, labels)
